# baseline (speedup 1.0000x reference)
import jax
import jax.numpy as jnp
from jax import lax
from jax.experimental import pallas as pl
from jax.experimental.pallas import tpu as pltpu
from jax.experimental.pallas import tpu_sc as plsc

BATCH = 4
SEQ_LEN = 8192
FEAT_DIM = 768
NW = 32
ROWS_W = SEQ_LEN // NW
R = 8
NBLK = ROWS_W // R
LANES = 16
CGRP = FEAT_DIM // LANES
UNROLL = 8
DEPTH = 3


def _body(in_hbm, pos_hbm, time_hbm, out_hbm, *scr):
    pbuf = scr[0:3]
    tbuf = scr[3:6]
    ibuf = scr[6:9]
    psem = scr[9:12]
    tsem = scr[12:15]
    isem = scr[15:18]
    osem = scr[18:21]

    wid = lax.axis_index("s") * 2 + lax.axis_index("c")
    base = wid * ROWS_W

    def wait_tbl(sem, vref):
        pltpu.make_async_copy(pos_hbm.at[pl.ds(0, R), :], vref, sem).wait()

    def wait_batch(sem, vref):
        pltpu.make_async_copy(in_hbm.at[:, pl.ds(0, R), :], vref, sem).wait()

    def vloop(body):
        @pl.loop(0, R)
        def _row(r):
            @plsc.parallel_loop(0, CGRP, unroll=UNROLL)
            def _col(c):
                body(r, pl.ds(c * LANES, LANES))

    def do_block(k, p, prefetch, wait_prev_out):
        pn = (p + 1) % DEPTH
        roff = base + k * R
        if prefetch:
            roffn = roff + R
            pltpu.async_copy(pos_hbm.at[pl.ds(roffn, R), :], pbuf[pn], psem[pn])
            pltpu.async_copy(time_hbm.at[pl.ds(roffn, R), :], tbuf[pn], tsem[pn])
            if wait_prev_out:
                pltpu.make_async_copy(
                    ibuf[pn], out_hbm.at[:, pl.ds(0, R), :], osem[pn]
                ).wait()
            pltpu.async_copy(
                in_hbm.at[:, pl.ds(roffn, R), :], ibuf[pn], isem[pn]
            )

        wait_tbl(psem[p], pbuf[p])
        wait_tbl(tsem[p], tbuf[p])
        pb = pbuf[p]
        tb = tbuf[p]
        ib = ibuf[p]
        wait_batch(isem[p], ib)

        for b in range(BATCH):
            if b == 0:
                def _accum(r, s):
                    v = pb[r, s] + tb[r, s]
                    pb[r, s] = v
                    plsc.addupdate(ib.at[0, r, s], v)
            else:
                def _accum(r, s, b=b):
                    plsc.addupdate(ib.at[b, r, s], pb[r, s])

            vloop(_accum)

        pltpu.async_copy(ib, out_hbm.at[:, pl.ds(roff, R), :], osem[p])

    pltpu.async_copy(pos_hbm.at[pl.ds(base, R), :], pbuf[0], psem[0])
    pltpu.async_copy(time_hbm.at[pl.ds(base, R), :], tbuf[0], tsem[0])
    pltpu.async_copy(in_hbm.at[:, pl.ds(base, R), :], ibuf[0], isem[0])

    do_block(0, 0, prefetch=True, wait_prev_out=False)
    do_block(1, 1, prefetch=True, wait_prev_out=False)

    @pl.loop(2, NBLK - 3, step=3)
    def _mid(k0):
        do_block(k0, 2, prefetch=True, wait_prev_out=True)
        do_block(k0 + 1, 0, prefetch=True, wait_prev_out=True)
        do_block(k0 + 2, 1, prefetch=True, wait_prev_out=True)

    do_block(NBLK - 3, 2, prefetch=True, wait_prev_out=True)
    do_block(NBLK - 2, 0, prefetch=True, wait_prev_out=True)
    do_block(NBLK - 1, 1, prefetch=False, wait_prev_out=False)

    for p in range(DEPTH):
        pltpu.make_async_copy(
            ibuf[p], out_hbm.at[:, pl.ds(0, R), :], osem[p]
        ).wait()


@jax.jit
def kernel(inputs, pos_table, time_table):
    mesh = plsc.VectorSubcoreMesh(core_axis_name="c", subcore_axis_name="s")
    return pl.kernel(
        _body,
        out_type=jax.ShapeDtypeStruct((BATCH, SEQ_LEN, FEAT_DIM), jnp.float32),
        mesh=mesh,
        compiler_params=pltpu.CompilerParams(use_tc_tiling_on_sc=True),
        scratch_types=(
            [pltpu.VMEM((R, FEAT_DIM), jnp.float32) for _ in range(6)]
            + [pltpu.VMEM((BATCH, R, FEAT_DIM), jnp.float32) for _ in range(3)]
            + [pltpu.SemaphoreType.DMA for _ in range(12)]
        ),
    )(inputs, pos_table, time_table)


# speedup vs baseline: 1.0098x; 1.0098x over previous
import jax
import jax.numpy as jnp
from jax import lax
from jax.experimental import pallas as pl
from jax.experimental.pallas import tpu as pltpu
from jax.experimental.pallas import tpu_sc as plsc

BATCH = 4
SEQ_LEN = 8192
FEAT_DIM = 768
NW = 32
ROWS_W = SEQ_LEN // NW
R = 8
NBLK = ROWS_W // R
LANES = 16
CGRP = FEAT_DIM // LANES
UNROLL = 8
DEPTH = 3


def _body(in_hbm, pos_hbm, time_hbm, out_hbm, *scr):
    pbuf = scr[0:3]
    tbuf = scr[3:6]
    ibuf = scr[6:9]
    psem = scr[9:12]
    tsem = scr[12:15]
    isem = scr[15:18]
    osem = scr[18:21]

    wid = lax.axis_index("s") * 2 + lax.axis_index("c")
    base = wid * ROWS_W

    def wait_tbl(sem, vref):
        pltpu.make_async_copy(pos_hbm.at[pl.ds(0, R), :], vref, sem).wait()

    def wait_batch(sem, vref):
        pltpu.make_async_copy(in_hbm.at[:, pl.ds(0, R), :], vref, sem).wait()

    def vloop(body):
        @pl.loop(0, R)
        def _row(r):
            @plsc.parallel_loop(0, CGRP, unroll=UNROLL)
            def _col(c):
                body(r, pl.ds(c * LANES, LANES))

    def do_block(k, p, prefetch, wait_prev_out):
        pn = (p + 1) % DEPTH
        roff = base + k * R
        wait_tbl(psem[p], pbuf[p])
        wait_tbl(tsem[p], tbuf[p])
        if prefetch:
            roffn = roff + R
            pltpu.async_copy(pos_hbm.at[pl.ds(roffn, R), :], pbuf[pn], psem[pn])
            pltpu.async_copy(time_hbm.at[pl.ds(roffn, R), :], tbuf[pn], tsem[pn])
            if wait_prev_out:
                pltpu.make_async_copy(
                    ibuf[pn], out_hbm.at[:, pl.ds(0, R), :], osem[pn]
                ).wait()
            pltpu.async_copy(
                in_hbm.at[:, pl.ds(roffn, R), :], ibuf[pn], isem[pn]
            )

        pb = pbuf[p]
        tb = tbuf[p]
        ib = ibuf[p]
        wait_batch(isem[p], ib)

        for b in range(BATCH):
            if b == 0:
                def _accum(r, s):
                    v = pb[r, s] + tb[r, s]
                    pb[r, s] = v
                    plsc.addupdate(ib.at[0, r, s], v)
            else:
                def _accum(r, s, b=b):
                    plsc.addupdate(ib.at[b, r, s], pb[r, s])

            vloop(_accum)

        pltpu.async_copy(ib, out_hbm.at[:, pl.ds(roff, R), :], osem[p])

    pltpu.async_copy(pos_hbm.at[pl.ds(base, R), :], pbuf[0], psem[0])
    pltpu.async_copy(time_hbm.at[pl.ds(base, R), :], tbuf[0], tsem[0])
    pltpu.async_copy(in_hbm.at[:, pl.ds(base, R), :], ibuf[0], isem[0])

    do_block(0, 0, prefetch=True, wait_prev_out=False)
    do_block(1, 1, prefetch=True, wait_prev_out=False)

    @pl.loop(2, NBLK - 3, step=3)
    def _mid(k0):
        do_block(k0, 2, prefetch=True, wait_prev_out=True)
        do_block(k0 + 1, 0, prefetch=True, wait_prev_out=True)
        do_block(k0 + 2, 1, prefetch=True, wait_prev_out=True)

    do_block(NBLK - 3, 2, prefetch=True, wait_prev_out=True)
    do_block(NBLK - 2, 0, prefetch=True, wait_prev_out=True)
    do_block(NBLK - 1, 1, prefetch=False, wait_prev_out=False)

    for p in range(DEPTH):
        pltpu.make_async_copy(
            ibuf[p], out_hbm.at[:, pl.ds(0, R), :], osem[p]
        ).wait()


@jax.jit
def kernel(inputs, pos_table, time_table):
    mesh = plsc.VectorSubcoreMesh(core_axis_name="c", subcore_axis_name="s")
    return pl.kernel(
        _body,
        out_type=jax.ShapeDtypeStruct((BATCH, SEQ_LEN, FEAT_DIM), jnp.float32),
        mesh=mesh,
        compiler_params=pltpu.CompilerParams(use_tc_tiling_on_sc=True),
        scratch_types=(
            [pltpu.VMEM((R, FEAT_DIM), jnp.float32) for _ in range(6)]
            + [pltpu.VMEM((BATCH, R, FEAT_DIM), jnp.float32) for _ in range(3)]
            + [pltpu.SemaphoreType.DMA for _ in range(12)]
        ),
    )(inputs, pos_table, time_table)


# speedup vs baseline: 1.0214x; 1.0115x over previous
import jax
import jax.numpy as jnp
from jax import lax
from jax.experimental import pallas as pl
from jax.experimental.pallas import tpu as pltpu
from jax.experimental.pallas import tpu_sc as plsc

BATCH = 4
SEQ_LEN = 8192
FEAT_DIM = 768
NW = 32
ROWS_W = SEQ_LEN // NW
R = 8
NBLK = ROWS_W // R
LANES = 16
CGRP = FEAT_DIM // LANES
UNROLL = 8
DEPTH = 3


def _body(in_hbm, pos_hbm, time_hbm, out_hbm, *scr):
    pbuf = scr[0:3]
    tbuf = scr[3:6]
    ibuf = scr[6:9]
    psem = scr[9:12]
    tsem = scr[12:15]
    isem = scr[15:18]
    osem = scr[18:21]

    wid = lax.axis_index("s") * 2 + lax.axis_index("c")
    base = wid * ROWS_W

    def wait_tbl(sem, vref):
        pltpu.make_async_copy(pos_hbm.at[pl.ds(0, R), :], vref, sem).wait()

    def wait_batch(sem, vref):
        pltpu.make_async_copy(in_hbm.at[:, pl.ds(0, R), :], vref, sem).wait()

    def vloop(body):
        @pl.loop(0, R)
        def _row(r):
            @plsc.parallel_loop(0, CGRP, unroll=UNROLL)
            def _col(c):
                body(r, pl.ds(c * LANES, LANES))

    def do_block(k, p, prefetch, wait_prev_out):
        pn = (p + 1) % DEPTH
        roff = base + k * R
        wait_tbl(psem[p], pbuf[p])
        wait_tbl(tsem[p], tbuf[p])
        if prefetch:
            roffn = roff + R
            pltpu.async_copy(pos_hbm.at[pl.ds(roffn, R), :], pbuf[pn], psem[pn])
            pltpu.async_copy(time_hbm.at[pl.ds(roffn, R), :], tbuf[pn], tsem[pn])
            if wait_prev_out:
                pltpu.make_async_copy(
                    ibuf[pn], out_hbm.at[:, pl.ds(0, R), :], osem[pn]
                ).wait()
            pltpu.async_copy(
                in_hbm.at[:, pl.ds(roffn, R), :], ibuf[pn], isem[pn]
            )

        pb = pbuf[p]
        tb = tbuf[p]
        ib = ibuf[p]
        wait_batch(isem[p], ib)

        for b in range(BATCH):
            if b == 0:
                def _accum(r, s):
                    v = pb[r, s] + tb[r, s]
                    pb[r, s] = v
                    plsc.addupdate(ib.at[0, r, s], v)
            else:
                def _accum(r, s, b=b):
                    plsc.addupdate(ib.at[b, r, s], pb[r, s])

            vloop(_accum)
            pltpu.async_copy(
                ib.at[b], out_hbm.at[b, pl.ds(roff, R), :], osem[p]
            )

    pltpu.async_copy(pos_hbm.at[pl.ds(base, R), :], pbuf[0], psem[0])
    pltpu.async_copy(time_hbm.at[pl.ds(base, R), :], tbuf[0], tsem[0])
    pltpu.async_copy(in_hbm.at[:, pl.ds(base, R), :], ibuf[0], isem[0])

    do_block(0, 0, prefetch=True, wait_prev_out=False)
    do_block(1, 1, prefetch=True, wait_prev_out=False)

    @pl.loop(2, NBLK - 3, step=3)
    def _mid(k0):
        do_block(k0, 2, prefetch=True, wait_prev_out=True)
        do_block(k0 + 1, 0, prefetch=True, wait_prev_out=True)
        do_block(k0 + 2, 1, prefetch=True, wait_prev_out=True)

    do_block(NBLK - 3, 2, prefetch=True, wait_prev_out=True)
    do_block(NBLK - 2, 0, prefetch=True, wait_prev_out=True)
    do_block(NBLK - 1, 1, prefetch=False, wait_prev_out=False)

    for p in range(DEPTH):
        pltpu.make_async_copy(
            ibuf[p], out_hbm.at[:, pl.ds(0, R), :], osem[p]
        ).wait()


@jax.jit
def kernel(inputs, pos_table, time_table):
    mesh = plsc.VectorSubcoreMesh(core_axis_name="c", subcore_axis_name="s")
    return pl.kernel(
        _body,
        out_type=jax.ShapeDtypeStruct((BATCH, SEQ_LEN, FEAT_DIM), jnp.float32),
        mesh=mesh,
        compiler_params=pltpu.CompilerParams(use_tc_tiling_on_sc=True),
        scratch_types=(
            [pltpu.VMEM((R, FEAT_DIM), jnp.float32) for _ in range(6)]
            + [pltpu.VMEM((BATCH, R, FEAT_DIM), jnp.float32) for _ in range(3)]
            + [pltpu.SemaphoreType.DMA for _ in range(12)]
        ),
    )(inputs, pos_table, time_table)
